# Initial kernel scaffold; baseline (speedup 1.0000x reference)
import jax
import jax.numpy as jnp
from jax.experimental import pallas as pl


def kernel(x, edge_index_0, edge_index_1, edge_index_2, node_types, W, b, W_out, b_out):
    raise NotImplementedError("write your pallas kernel here")


# speedup vs baseline: 7.0106x; 7.0106x over previous
import functools

import jax
import jax.numpy as jnp
from jax import lax
from jax.experimental import pallas as pl
from jax.experimental.pallas import tpu as pltpu
from jax.experimental.pallas import tpu_sc as plsc

N = 10000
E = 320000
D = 128
NTYPES = 8
DEPTH = 3

NC = 2
NS = 16
CH = 80
EPT = E // (NC * NS)
NCH = EPT // CH
RPT = 624
TAIL = N - NS * RPT

BN = 400
NB = N // BN


def _sc_layer_body(src_hbm, dst_hbm, h_hbm, out_hbm, src_v, dst_v, rows_v,
                   agg_sh):
    cid = lax.axis_index("c")
    sid = lax.axis_index("s")

    def _zr(r, _):
        for g in range(D // 16):
            rows_v[r, pl.ds(g * 16, 16)] = jnp.zeros((16,), jnp.float32)
        return _
    lax.fori_loop(0, CH, _zr, 0)
    base = sid * RPT
    for k in range(RPT // CH):
        pltpu.sync_copy(rows_v, agg_sh.at[pl.ds(base + k * CH, CH)])
    rem = RPT - (RPT // CH) * CH
    if rem:
        pltpu.sync_copy(rows_v.at[pl.ds(0, rem)],
                        agg_sh.at[pl.ds(base + (RPT // CH) * CH, rem)])

    @pl.when(sid == NS - 1)
    def _():
        pltpu.sync_copy(rows_v.at[pl.ds(0, TAIL)],
                        agg_sh.at[pl.ds(NS * RPT, TAIL)])

    pltpu.sync_copy(src_hbm.at[cid, sid], src_v)
    pltpu.sync_copy(dst_hbm.at[cid, sid], dst_v)

    plsc.subcore_barrier()

    def _chunk(j, _):
        pltpu.sync_copy(h_hbm.at[src_v.at[j]], rows_v)
        pltpu.sync_copy(rows_v, agg_sh.at[dst_v.at[j]], add=True)
        return _
    lax.fori_loop(0, NCH, _chunk, 0)

    plsc.subcore_barrier()

    pltpu.sync_copy(agg_sh.at[pl.ds(sid * RPT, RPT)],
                    out_hbm.at[cid, pl.ds(sid * RPT, RPT)])

    @pl.when(sid == NS - 1)
    def _():
        pltpu.sync_copy(agg_sh.at[pl.ds(NS * RPT, TAIL)],
                        out_hbm.at[cid, pl.ds(NS * RPT, TAIL)])


@functools.partial(jax.jit, static_argnums=())
def _sc_layer(src_idx, dst_idx, h):
    k = pl.kernel(
        _sc_layer_body,
        out_type=jax.ShapeDtypeStruct((NC, N, D), jnp.float32),
        mesh=plsc.VectorSubcoreMesh(core_axis_name="c", subcore_axis_name="s"),
        scratch_types=[
            pltpu.VMEM((NCH, CH), jnp.int32),
            pltpu.VMEM((NCH, CH), jnp.int32),
            pltpu.VMEM((CH, D), jnp.float32),
            pltpu.VMEM_SHARED((N, D), jnp.float32),
        ],
    )
    return k(src_idx, dst_idx, h)


def _tc_mid_body(parts_ref, types_ref, wcat_ref, bcat_ref, out_ref):
    agg = parts_ref[0] + parts_ref[1]
    y = jnp.dot(agg, wcat_ref[...], preferred_element_type=jnp.float32)
    y = y + bcat_ref[...]
    t = types_ref[...]
    acc = y[:, 0:D]
    for tt in range(1, NTYPES):
        acc = jnp.where(t == tt, y[:, tt * D:(tt + 1) * D], acc)
    out_ref[...] = jnp.maximum(acc, 0.0)


def _tc_mid(parts, types3, wcat, bcat):
    return pl.pallas_call(
        _tc_mid_body,
        grid=(NB,),
        in_specs=[
            pl.BlockSpec((NC, BN, D), lambda i: (0, i, 0)),
            pl.BlockSpec((BN, 1), lambda i: (i, 0)),
            pl.BlockSpec((D, NTYPES * D), lambda i: (0, 0)),
            pl.BlockSpec((1, NTYPES * D), lambda i: (0, 0)),
        ],
        out_specs=pl.BlockSpec((BN, D), lambda i: (i, 0)),
        out_shape=jax.ShapeDtypeStruct((N, D), jnp.float32),
    )(parts, types3, wcat, bcat)


def _tc_final_body(parts_ref, types_ref, wc_ref, bc_ref, out_ref):
    agg = parts_ref[0] + parts_ref[1]
    y = jnp.dot(agg, wc_ref[...], preferred_element_type=jnp.float32)
    y = y + bc_ref[...]
    t = types_ref[...]
    onehot = (t == lax.broadcasted_iota(jnp.int32, (1, NTYPES), 1))
    out_ref[...] = jnp.sum(jnp.where(onehot, y, 0.0), axis=1, keepdims=True)


def _tc_final(parts, types3, wc, bc):
    return pl.pallas_call(
        _tc_final_body,
        grid=(NB,),
        in_specs=[
            pl.BlockSpec((NC, BN, D), lambda i: (0, i, 0)),
            pl.BlockSpec((BN, 1), lambda i: (i, 0)),
            pl.BlockSpec((D, NTYPES), lambda i: (0, 0)),
            pl.BlockSpec((1, NTYPES), lambda i: (0, 0)),
        ],
        out_specs=pl.BlockSpec((BN, 1), lambda i: (i, 0)),
        out_shape=jax.ShapeDtypeStruct((N, 1), jnp.float32),
    )(parts, types3, wc, bc)


def kernel(x, edge_index_0, edge_index_1, edge_index_2, node_types, W, b,
           W_out, b_out):
    types2 = node_types.reshape(N, 1)
    wcat = jnp.transpose(W, (1, 0, 2)).reshape(D, NTYPES * D)
    bcat = b.reshape(1, NTYPES * D)
    wc = jnp.transpose((W @ W_out)[:, :, 0], (1, 0))
    bc = (b @ W_out).reshape(1, NTYPES) + b_out[0]

    h = x
    for i, ei in enumerate((edge_index_0, edge_index_1, edge_index_2)):
        src = ei[0].reshape(NC, NS, NCH, CH)
        dst = ei[1].reshape(NC, NS, NCH, CH)
        parts = _sc_layer(src, dst, h)
        if i != DEPTH - 1:
            h = _tc_mid(parts, types2, wcat, bcat)
        else:
            out = _tc_final(parts, types2, wc, bc)
    return out.reshape(N)


# speedup vs baseline: 8.8048x; 1.2559x over previous
import functools

import jax
import jax.numpy as jnp
from jax import lax
from jax.experimental import pallas as pl
from jax.experimental.pallas import tpu as pltpu
from jax.experimental.pallas import tpu_sc as plsc

N = 10000
E = 320000
D = 128
NTYPES = 8
DEPTH = 3

NC = 2
NS = 16
CH = 80
EPT = E // (NC * NS)
NCH = EPT // CH
NBLK = 5
CPB = NCH // NBLK
RPT = 624
TAIL = N - NS * RPT

BN = 400
NB = N // BN


def _sc_layer_body(src_hbm, dst_hbm, h_hbm, out_hbm, src_v, dst_v, rows_v,
                   agg_sh, gsem, isem_s, isem_d):
    cid = lax.axis_index("c")
    sid = lax.axis_index("s")

    def _zr(r, _):
        for g in range(D // 16):
            rows_v[0, r, pl.ds(g * 16, 16)] = jnp.zeros((16,), jnp.float32)
        return _
    lax.fori_loop(0, CH, _zr, 0)
    base = sid * RPT
    ZCH = 48
    for k in range(RPT // ZCH):
        pltpu.sync_copy(rows_v.at[0, pl.ds(0, ZCH)],
                        agg_sh.at[pl.ds(base + k * ZCH, ZCH)])

    @pl.when(sid == NS - 1)
    def _():
        pltpu.sync_copy(rows_v.at[0, pl.ds(0, TAIL)],
                        agg_sh.at[pl.ds(NS * RPT, TAIL)])

    pltpu.sync_copy(src_hbm.at[cid, sid, 0], src_v.at[0])
    pltpu.sync_copy(dst_hbm.at[cid, sid, 0], dst_v.at[0])

    plsc.subcore_barrier()

    pltpu.async_copy(h_hbm.at[src_v.at[0, 0]], rows_v.at[0], gsem)

    for blk in range(NBLK):
        bb = blk % 2
        if blk + 1 < NBLK:
            pltpu.async_copy(src_hbm.at[cid, sid, blk + 1],
                             src_v.at[1 - bb], isem_s)
            pltpu.async_copy(dst_hbm.at[cid, sid, blk + 1],
                             dst_v.at[1 - bb], isem_d)

        def _chunk(j, carry, bb=bb, blk=blk):
            buf = lax.rem(blk * CPB + j, 2)
            pltpu.make_async_copy(h_hbm.at[src_v.at[bb, j]],
                                  rows_v.at[buf], gsem).wait()

            @pl.when(j + 1 < CPB)
            def _prefetch():
                pltpu.async_copy(h_hbm.at[src_v.at[bb, j + 1]],
                                 rows_v.at[1 - buf], gsem)

            pltpu.sync_copy(rows_v.at[buf], agg_sh.at[dst_v.at[bb, j]],
                            add=True)
            return carry
        lax.fori_loop(0, CPB, _chunk, 0)

        if blk + 1 < NBLK:
            pltpu.make_async_copy(src_hbm.at[cid, sid, blk + 1],
                                  src_v.at[1 - bb], isem_s).wait()
            pltpu.make_async_copy(dst_hbm.at[cid, sid, blk + 1],
                                  dst_v.at[1 - bb], isem_d).wait()
            nbuf = lax.rem((blk + 1) * CPB, 2)
            pltpu.async_copy(h_hbm.at[src_v.at[1 - bb, 0]],
                             rows_v.at[nbuf], gsem)

    plsc.subcore_barrier()

    pltpu.sync_copy(agg_sh.at[pl.ds(sid * RPT, RPT)],
                    out_hbm.at[cid, pl.ds(sid * RPT, RPT)])

    @pl.when(sid == NS - 1)
    def _():
        pltpu.sync_copy(agg_sh.at[pl.ds(NS * RPT, TAIL)],
                        out_hbm.at[cid, pl.ds(NS * RPT, TAIL)])


@functools.partial(jax.jit, static_argnums=())
def _sc_layer(src_idx, dst_idx, h):
    k = pl.kernel(
        _sc_layer_body,
        out_type=jax.ShapeDtypeStruct((NC, N, D), jnp.float32),
        mesh=plsc.VectorSubcoreMesh(core_axis_name="c", subcore_axis_name="s"),
        scratch_types=[
            pltpu.VMEM((2, CPB, CH), jnp.int32),
            pltpu.VMEM((2, CPB, CH), jnp.int32),
            pltpu.VMEM((2, CH, D), jnp.float32),
            pltpu.VMEM_SHARED((N, D), jnp.float32),
            pltpu.SemaphoreType.DMA,
            pltpu.SemaphoreType.DMA,
            pltpu.SemaphoreType.DMA,
        ],
    )
    return k(src_idx, dst_idx, h)


def _tc_mid_body(parts_ref, types_ref, wcat_ref, bcat_ref, out_ref):
    agg = parts_ref[0] + parts_ref[1]
    y = jnp.dot(agg, wcat_ref[...], preferred_element_type=jnp.float32)
    y = y + bcat_ref[...]
    t = types_ref[...]
    acc = y[:, 0:D]
    for tt in range(1, NTYPES):
        acc = jnp.where(t == tt, y[:, tt * D:(tt + 1) * D], acc)
    out_ref[...] = jnp.maximum(acc, 0.0)


def _tc_mid(parts, types3, wcat, bcat):
    return pl.pallas_call(
        _tc_mid_body,
        grid=(NB,),
        in_specs=[
            pl.BlockSpec((NC, BN, D), lambda i: (0, i, 0)),
            pl.BlockSpec((BN, 1), lambda i: (i, 0)),
            pl.BlockSpec((D, NTYPES * D), lambda i: (0, 0)),
            pl.BlockSpec((1, NTYPES * D), lambda i: (0, 0)),
        ],
        out_specs=pl.BlockSpec((BN, D), lambda i: (i, 0)),
        out_shape=jax.ShapeDtypeStruct((N, D), jnp.float32),
    )(parts, types3, wcat, bcat)


def _tc_final_body(parts_ref, types_ref, wc_ref, bc_ref, out_ref):
    agg = parts_ref[0] + parts_ref[1]
    y = jnp.dot(agg, wc_ref[...], preferred_element_type=jnp.float32)
    y = y + bc_ref[...]
    t = types_ref[...]
    onehot = (t == lax.broadcasted_iota(jnp.int32, (1, NTYPES), 1))
    out_ref[...] = jnp.sum(jnp.where(onehot, y, 0.0), axis=1, keepdims=True)


def _tc_final(parts, types3, wc, bc):
    return pl.pallas_call(
        _tc_final_body,
        grid=(NB,),
        in_specs=[
            pl.BlockSpec((NC, BN, D), lambda i: (0, i, 0)),
            pl.BlockSpec((BN, 1), lambda i: (i, 0)),
            pl.BlockSpec((D, NTYPES), lambda i: (0, 0)),
            pl.BlockSpec((1, NTYPES), lambda i: (0, 0)),
        ],
        out_specs=pl.BlockSpec((BN, 1), lambda i: (i, 0)),
        out_shape=jax.ShapeDtypeStruct((N, 1), jnp.float32),
    )(parts, types3, wc, bc)


def kernel(x, edge_index_0, edge_index_1, edge_index_2, node_types, W, b,
           W_out, b_out):
    types2 = node_types.reshape(N, 1)
    wcat = jnp.transpose(W, (1, 0, 2)).reshape(D, NTYPES * D)
    bcat = b.reshape(1, NTYPES * D)
    wc = jnp.transpose((W @ W_out)[:, :, 0], (1, 0))
    bc = (b @ W_out).reshape(1, NTYPES) + b_out[0]

    h = x
    for i, ei in enumerate((edge_index_0, edge_index_1, edge_index_2)):
        src = ei[0].reshape(NC, NS, NBLK, CPB, CH)
        dst = ei[1].reshape(NC, NS, NBLK, CPB, CH)
        parts = _sc_layer(src, dst, h)
        if i != DEPTH - 1:
            h = _tc_mid(parts, types2, wcat, bcat)
        else:
            out = _tc_final(parts, types2, wc, bc)
    return out.reshape(N)


# speedup vs baseline: 9.1764x; 1.0422x over previous
import functools

import jax
import jax.numpy as jnp
from jax import lax
from jax.experimental import pallas as pl
from jax.experimental.pallas import tpu as pltpu
from jax.experimental.pallas import tpu_sc as plsc

N = 10000
E = 320000
D = 128
NTYPES = 8
DEPTH = 3

NC = 2
NS = 16
DH = D // NC
CH = 100
EPT = E // NS
NCH = EPT // CH
CPB = 20
NBLK = NCH // CPB
NBUF = 4
RPT = 624
TAIL = N - NS * RPT
ZCH = 48

BN = 400
NB = N // BN


def _sc_layer_body(src_hbm, dst_hbm, h_hbm, out_hbm, src_v, dst_v, rows_v,
                   agg_sh, gsems, ssems, isems):
    cid = lax.axis_index("c")
    sid = lax.axis_index("s")

    def issue_gather(idxbuf, row, buf):
        pltpu.async_copy(h_hbm.at[cid].at[src_v.at[idxbuf, row]],
                         rows_v.at[buf], gsems.at[buf])

    def wait_gather(buf):
        pltpu.make_async_copy(h_hbm.at[cid].at[src_v.at[0, 0]],
                              rows_v.at[buf], gsems.at[buf]).wait()

    def issue_scatter(idxbuf, row, buf):
        pltpu.async_copy(rows_v.at[buf], agg_sh.at[dst_v.at[idxbuf, row]],
                         ssems.at[buf], add=True)

    def wait_scatter(buf):
        pltpu.make_async_copy(rows_v.at[buf], agg_sh.at[dst_v.at[0, 0]],
                              ssems.at[buf]).wait()

    def chunk(bb, row, buf, ssem_wait=True, pf=None):
        wait_gather(buf)
        issue_scatter(bb, row, buf)
        if ssem_wait:
            wait_scatter((buf + 2) % NBUF)
        if pf is not None:
            issue_gather(pf[0], pf[1], (buf + 2) % NBUF)

    def _zr(r, carry):
        for g in range(DH // 16):
            rows_v[0, r, pl.ds(g * 16, 16)] = jnp.zeros((16,), jnp.float32)
        return carry
    lax.fori_loop(0, CH, _zr, 0)
    base = sid * RPT
    for k in range(RPT // ZCH):
        pltpu.sync_copy(rows_v.at[0, pl.ds(0, ZCH)],
                        agg_sh.at[pl.ds(base + k * ZCH, ZCH)])

    @pl.when(sid == NS - 1)
    def _():
        pltpu.sync_copy(rows_v.at[0, pl.ds(0, TAIL)],
                        agg_sh.at[pl.ds(NS * RPT, TAIL)])

    pltpu.sync_copy(src_hbm.at[sid, 0], src_v.at[0])
    pltpu.sync_copy(dst_hbm.at[sid, 0], dst_v.at[0])

    plsc.subcore_barrier()

    issue_gather(0, 0, 0)
    issue_gather(0, 1, 1)

    for blk in range(NBLK):
        bb = blk % 2
        if blk > 0:
            wait_scatter(2)
            wait_scatter(3)
        if blk + 1 < NBLK:
            pltpu.async_copy(src_hbm.at[sid, blk + 1], src_v.at[1 - bb],
                             isems.at[0])
            pltpu.async_copy(dst_hbm.at[sid, blk + 1], dst_v.at[1 - bb],
                             isems.at[1])

        chunk(bb, 0, 0, ssem_wait=False, pf=(bb, 2, 2))
        chunk(bb, 1, 1, ssem_wait=False, pf=(bb, 3, 3))
        chunk(bb, 2, 2, pf=(bb, 4, 0))
        chunk(bb, 3, 3, pf=(bb, 5, 1))

        def _grp(k, carry, bb=bb):
            for u in range(4):
                chunk(bb, 4 * k + u, u, pf=(bb, 4 * k + u + 2, (u + 2) % 4))
            return carry
        lax.fori_loop(1, CPB // 4 - 1, _grp, 0)

        chunk(bb, CPB - 4, 0, pf=(bb, CPB - 2, 2))
        chunk(bb, CPB - 3, 1, pf=(bb, CPB - 1, 3))
        if blk + 1 < NBLK:
            pltpu.make_async_copy(src_hbm.at[sid, blk + 1],
                                  src_v.at[1 - bb], isems.at[0]).wait()
            pltpu.make_async_copy(dst_hbm.at[sid, blk + 1],
                                  dst_v.at[1 - bb], isems.at[1]).wait()
            chunk(bb, CPB - 2, 2, pf=(1 - bb, 0, 0))
            chunk(bb, CPB - 1, 3, pf=(1 - bb, 1, 1))
        else:
            chunk(bb, CPB - 2, 2)
            chunk(bb, CPB - 1, 3)

    wait_scatter(2)
    wait_scatter(3)

    plsc.subcore_barrier()

    pltpu.sync_copy(agg_sh.at[pl.ds(sid * RPT, RPT)],
                    out_hbm.at[cid, pl.ds(sid * RPT, RPT)])

    @pl.when(sid == NS - 1)
    def _():
        pltpu.sync_copy(agg_sh.at[pl.ds(NS * RPT, TAIL)],
                        out_hbm.at[cid, pl.ds(NS * RPT, TAIL)])


@functools.partial(jax.jit, static_argnums=())
def _sc_layer(src_idx, dst_idx, h):
    k = pl.kernel(
        _sc_layer_body,
        out_type=jax.ShapeDtypeStruct((NC, N, DH), jnp.float32),
        mesh=plsc.VectorSubcoreMesh(core_axis_name="c", subcore_axis_name="s"),
        compiler_params=pltpu.CompilerParams(use_tc_tiling_on_sc=False),
        scratch_types=[
            pltpu.VMEM((2, CPB, CH), jnp.int32),
            pltpu.VMEM((2, CPB, CH), jnp.int32),
            pltpu.VMEM((NBUF, CH, DH), jnp.float32),
            pltpu.VMEM_SHARED((N, DH), jnp.float32),
            pltpu.SemaphoreType.DMA((NBUF,)),
            pltpu.SemaphoreType.DMA((NBUF,)),
            pltpu.SemaphoreType.DMA((2,)),
        ],
    )
    return k(src_idx, dst_idx, h)


def _tc_mid_body(parts_ref, types_ref, wcat_ref, bcat_ref, out_ref):
    agg = jnp.concatenate([parts_ref[0], parts_ref[1]], axis=1)
    y = jnp.dot(agg, wcat_ref[...], preferred_element_type=jnp.float32)
    y = y + bcat_ref[...]
    t = types_ref[...]
    acc = y[:, 0:D]
    for tt in range(1, NTYPES):
        acc = jnp.where(t == tt, y[:, tt * D:(tt + 1) * D], acc)
    res = jnp.maximum(acc, 0.0)
    out_ref[0] = res[:, 0:DH]
    out_ref[1] = res[:, DH:D]


def _tc_mid(parts, types2, wcat, bcat):
    return pl.pallas_call(
        _tc_mid_body,
        grid=(NB,),
        in_specs=[
            pl.BlockSpec((NC, BN, DH), lambda i: (0, i, 0)),
            pl.BlockSpec((BN, 1), lambda i: (i, 0)),
            pl.BlockSpec((D, NTYPES * D), lambda i: (0, 0)),
            pl.BlockSpec((1, NTYPES * D), lambda i: (0, 0)),
        ],
        out_specs=pl.BlockSpec((NC, BN, DH), lambda i: (0, i, 0)),
        out_shape=jax.ShapeDtypeStruct((NC, N, DH), jnp.float32),
    )(parts, types2, wcat, bcat)


def _tc_final_body(parts_ref, types_ref, wc_ref, bc_ref, out_ref):
    agg = jnp.concatenate([parts_ref[0], parts_ref[1]], axis=1)
    y = jnp.dot(agg, wc_ref[...], preferred_element_type=jnp.float32)
    y = y + bc_ref[...]
    t = types_ref[...]
    onehot = (t == lax.broadcasted_iota(jnp.int32, (1, NTYPES), 1))
    out_ref[...] = jnp.sum(jnp.where(onehot, y, 0.0), axis=1, keepdims=True)


def _tc_final(parts, types2, wc, bc):
    return pl.pallas_call(
        _tc_final_body,
        grid=(NB,),
        in_specs=[
            pl.BlockSpec((NC, BN, DH), lambda i: (0, i, 0)),
            pl.BlockSpec((BN, 1), lambda i: (i, 0)),
            pl.BlockSpec((D, NTYPES), lambda i: (0, 0)),
            pl.BlockSpec((1, NTYPES), lambda i: (0, 0)),
        ],
        out_specs=pl.BlockSpec((BN, 1), lambda i: (i, 0)),
        out_shape=jax.ShapeDtypeStruct((N, 1), jnp.float32),
    )(parts, types2, wc, bc)


def kernel(x, edge_index_0, edge_index_1, edge_index_2, node_types, W, b,
           W_out, b_out):
    types2 = node_types.reshape(N, 1)
    wcat = jnp.transpose(W, (1, 0, 2)).reshape(D, NTYPES * D)
    bcat = b.reshape(1, NTYPES * D)
    wc = jnp.transpose((W @ W_out)[:, :, 0], (1, 0))
    bc = (b @ W_out).reshape(1, NTYPES) + b_out[0]

    h = x.reshape(N, NC, DH).transpose(1, 0, 2)
    for i, ei in enumerate((edge_index_0, edge_index_1, edge_index_2)):
        src = ei[0].reshape(NS, NBLK, CPB, CH)
        dst = ei[1].reshape(NS, NBLK, CPB, CH)
        parts = _sc_layer(src, dst, h)
        if i != DEPTH - 1:
            h = _tc_mid(parts, types2, wcat, bcat)
        else:
            out = _tc_final(parts, types2, wc, bc)
    return out.reshape(N)
